# baseline (device time: 8394 ns/iter reference)
import jax
import jax.numpy as jnp
from jax import lax
from jax.experimental import pallas as pl
from jax.experimental.pallas import tpu as pltpu

N_DEV = 4


def kernel(x):
    m_per, n = x.shape

    def body(x_ref, out_ref, total_ref, recv_buf, send_sems, recv_sems):
        my_pos = lax.axis_index("i")

        barrier_sem = pltpu.get_barrier_semaphore()
        for p in range(1, N_DEV):
            pl.semaphore_signal(
                barrier_sem, inc=1,
                device_id=((my_pos + p) % N_DEV,),
                device_id_type=pl.DeviceIdType.MESH,
            )

        x = x_ref[:, :]
        t = x
        rows = m_per
        while rows > 1:
            half = rows // 2
            t = t[:half, :] * t[half:rows, :]
            rows = half
        total_ref[:, :] = t

        pl.semaphore_wait(barrier_sem, N_DEV - 1)

        for d in range(1, N_DEV):
            @pl.when(my_pos + d < N_DEV)
            def _(d=d):
                pltpu.make_async_remote_copy(
                    src_ref=total_ref,
                    dst_ref=recv_buf.at[d],
                    send_sem=send_sems.at[d],
                    recv_sem=recv_sems.at[d],
                    device_id=((my_pos + d) % N_DEV,),
                    device_id_type=pl.DeviceIdType.MESH,
                ).start()

        G = 32
        B = m_per // G
        v = x.reshape(G, B, n)
        k = 1
        while k < B:
            shifted = jnp.concatenate(
                [jnp.ones((G, k, n), v.dtype), v[:, : B - k, :]], axis=1
            )
            v = v * shifted
            k *= 2

        c = v[:, B - 1, :]
        k = 1
        while k < G:
            c = c * jnp.concatenate(
                [jnp.ones((k, n), c.dtype), c[: G - k, :]], axis=0
            )
            k *= 2
        excl = jnp.concatenate([jnp.ones((1, n), c.dtype), c[: G - 1, :]], axis=0)

        for d in range(1, N_DEV):
            @pl.when(my_pos >= d)
            def _(d=d):
                pltpu.make_async_remote_copy(
                    src_ref=total_ref,
                    dst_ref=recv_buf.at[d],
                    send_sem=send_sems.at[d],
                    recv_sem=recv_sems.at[d],
                    device_id=((my_pos - d) % N_DEV,),
                    device_id_type=pl.DeviceIdType.MESH,
                ).wait_recv()

        prefix = jnp.ones((1, n), jnp.float32)
        for d in range(1, N_DEV):
            prefix = prefix * jnp.where(my_pos >= d, recv_buf[d], 1.0)
        scale = excl * prefix
        out_ref[:, :] = (v * scale[:, None, :]).reshape(m_per, n)

        for d in range(1, N_DEV):
            @pl.when(my_pos + d < N_DEV)
            def _(d=d):
                pltpu.make_async_remote_copy(
                    src_ref=total_ref,
                    dst_ref=recv_buf.at[d],
                    send_sem=send_sems.at[d],
                    recv_sem=recv_sems.at[d],
                    device_id=((my_pos + d) % N_DEV,),
                    device_id_type=pl.DeviceIdType.MESH,
                ).wait_send()

    return pl.pallas_call(
        body,
        out_shape=jax.ShapeDtypeStruct((m_per, n), x.dtype),
        in_specs=[pl.BlockSpec(memory_space=pltpu.VMEM)],
        out_specs=pl.BlockSpec(memory_space=pltpu.VMEM),
        scratch_shapes=[
            pltpu.VMEM((1, n), x.dtype),
            pltpu.VMEM((N_DEV, 1, n), x.dtype),
            pltpu.SemaphoreType.DMA((N_DEV,)),
            pltpu.SemaphoreType.DMA((N_DEV,)),
        ],
        compiler_params=pltpu.CompilerParams(collective_id=0),
    )(x)


# device time: 7918 ns/iter; 1.0601x vs baseline; 1.0601x over previous
import jax
import jax.numpy as jnp
from jax import lax
from jax.experimental import pallas as pl
from jax.experimental.pallas import tpu as pltpu

N_DEV = 4


def kernel(x):
    m_per, n = x.shape

    def body(x_ref, out_ref, total_ref, recv_buf, send_sems, recv_sems):
        my_pos = lax.axis_index("i")

        barrier_sem = pltpu.get_barrier_semaphore()
        for p in range(1, N_DEV):
            pl.semaphore_signal(
                barrier_sem, inc=1,
                device_id=((my_pos + p) % N_DEV,),
                device_id_type=pl.DeviceIdType.MESH,
            )

        x = x_ref[:, :]
        t = x
        rows = m_per
        while rows > 1:
            half = rows // 2
            t = t[:half, :] * t[half:rows, :]
            rows = half
        total_ref[:, :] = t

        v = x
        k = 1
        while k < 4:
            shifted = jnp.concatenate(
                [jnp.ones((k, n), v.dtype), v[: m_per - k, :]], axis=0
            )
            v = v * shifted
            k *= 2

        pl.semaphore_wait(barrier_sem, N_DEV - 1)

        for d in range(1, N_DEV):
            @pl.when(my_pos + d < N_DEV)
            def _(d=d):
                pltpu.make_async_remote_copy(
                    src_ref=total_ref,
                    dst_ref=recv_buf.at[d],
                    send_sem=send_sems.at[d],
                    recv_sem=recv_sems.at[d],
                    device_id=((my_pos + d) % N_DEV,),
                    device_id_type=pl.DeviceIdType.MESH,
                ).start()

        while k < m_per:
            shifted = jnp.concatenate(
                [jnp.ones((k, n), v.dtype), v[: m_per - k, :]], axis=0
            )
            v = v * shifted
            k *= 2

        for d in range(1, N_DEV):
            @pl.when(my_pos >= d)
            def _(d=d):
                pltpu.make_async_remote_copy(
                    src_ref=total_ref,
                    dst_ref=recv_buf.at[d],
                    send_sem=send_sems.at[d],
                    recv_sem=recv_sems.at[d],
                    device_id=((my_pos - d) % N_DEV,),
                    device_id_type=pl.DeviceIdType.MESH,
                ).wait_recv()

        prefix = jnp.ones((1, n), jnp.float32)
        for d in range(1, N_DEV):
            prefix = prefix * jnp.where(my_pos >= d, recv_buf[d], 1.0)
        out_ref[:, :] = v * prefix

        for d in range(1, N_DEV):
            @pl.when(my_pos + d < N_DEV)
            def _(d=d):
                pltpu.make_async_remote_copy(
                    src_ref=total_ref,
                    dst_ref=recv_buf.at[d],
                    send_sem=send_sems.at[d],
                    recv_sem=recv_sems.at[d],
                    device_id=((my_pos + d) % N_DEV,),
                    device_id_type=pl.DeviceIdType.MESH,
                ).wait_send()

    return pl.pallas_call(
        body,
        out_shape=jax.ShapeDtypeStruct((m_per, n), x.dtype),
        in_specs=[pl.BlockSpec(memory_space=pltpu.VMEM)],
        out_specs=pl.BlockSpec(memory_space=pltpu.VMEM),
        scratch_shapes=[
            pltpu.VMEM((1, n), x.dtype),
            pltpu.VMEM((N_DEV, 1, n), x.dtype),
            pltpu.SemaphoreType.DMA((N_DEV,)),
            pltpu.SemaphoreType.DMA((N_DEV,)),
        ],
        compiler_params=pltpu.CompilerParams(collective_id=0),
    )(x)


# device time: 7445 ns/iter; 1.1275x vs baseline; 1.0635x over previous
import jax
import jax.numpy as jnp
from jax import lax
from jax.experimental import pallas as pl
from jax.experimental.pallas import tpu as pltpu

N_DEV = 4


def kernel(x):
    m_per, n = x.shape

    def body(x_ref, out_ref, total_ref, recv_buf, send_sems, recv_sems):
        my_pos = lax.axis_index("i")

        barrier_sem = pltpu.get_barrier_semaphore()
        for p in range(1, N_DEV):
            pl.semaphore_signal(
                barrier_sem, inc=1,
                device_id=((my_pos + p) % N_DEV,),
                device_id_type=pl.DeviceIdType.MESH,
            )

        x = x_ref[:, :]
        t = x
        rows = m_per
        while rows > 1:
            half = rows // 2
            t = t[:half, :] * t[half:rows, :]
            rows = half
        total_ref[:, :] = t

        v = x
        k = 1
        while k < 16:
            shifted = jnp.concatenate(
                [jnp.ones((k, n), v.dtype), v[: m_per - k, :]], axis=0
            )
            v = v * shifted
            k *= 2

        pl.semaphore_wait(barrier_sem, N_DEV - 1)

        for d in range(1, N_DEV):
            @pl.when(my_pos + d < N_DEV)
            def _(d=d):
                pltpu.make_async_remote_copy(
                    src_ref=total_ref,
                    dst_ref=recv_buf.at[d],
                    send_sem=send_sems.at[d],
                    recv_sem=recv_sems.at[d],
                    device_id=((my_pos + d) % N_DEV,),
                    device_id_type=pl.DeviceIdType.MESH,
                ).start()

        while k < m_per:
            shifted = jnp.concatenate(
                [jnp.ones((k, n), v.dtype), v[: m_per - k, :]], axis=0
            )
            v = v * shifted
            k *= 2

        for d in range(1, N_DEV):
            @pl.when(my_pos >= d)
            def _(d=d):
                pltpu.make_async_remote_copy(
                    src_ref=total_ref,
                    dst_ref=recv_buf.at[d],
                    send_sem=send_sems.at[d],
                    recv_sem=recv_sems.at[d],
                    device_id=((my_pos - d) % N_DEV,),
                    device_id_type=pl.DeviceIdType.MESH,
                ).wait_recv()

        prefix = jnp.ones((1, n), jnp.float32)
        for d in range(1, N_DEV):
            prefix = prefix * jnp.where(my_pos >= d, recv_buf[d], 1.0)
        out_ref[:, :] = v * prefix

        for d in range(1, N_DEV):
            @pl.when(my_pos + d < N_DEV)
            def _(d=d):
                pltpu.make_async_remote_copy(
                    src_ref=total_ref,
                    dst_ref=recv_buf.at[d],
                    send_sem=send_sems.at[d],
                    recv_sem=recv_sems.at[d],
                    device_id=((my_pos + d) % N_DEV,),
                    device_id_type=pl.DeviceIdType.MESH,
                ).wait_send()

    return pl.pallas_call(
        body,
        out_shape=jax.ShapeDtypeStruct((m_per, n), x.dtype),
        in_specs=[pl.BlockSpec(memory_space=pltpu.VMEM)],
        out_specs=pl.BlockSpec(memory_space=pltpu.VMEM),
        scratch_shapes=[
            pltpu.VMEM((1, n), x.dtype),
            pltpu.VMEM((N_DEV, 1, n), x.dtype),
            pltpu.SemaphoreType.DMA((N_DEV,)),
            pltpu.SemaphoreType.DMA((N_DEV,)),
        ],
        compiler_params=pltpu.CompilerParams(collective_id=0),
    )(x)


# device time: 7286 ns/iter; 1.1521x vs baseline; 1.0218x over previous
import jax
import jax.numpy as jnp
from jax import lax
from jax.experimental import pallas as pl
from jax.experimental.pallas import tpu as pltpu

N_DEV = 4


def kernel(x):
    m_per, n = x.shape

    def body(x_ref, out_ref, total_ref, recv_buf, send_sems, recv_sems):
        my_pos = lax.axis_index("i")

        barrier_sem = pltpu.get_barrier_semaphore()
        for p in range(1, N_DEV):
            @pl.when(my_pos >= p)
            def _(p=p):
                pl.semaphore_signal(
                    barrier_sem, inc=1,
                    device_id=((my_pos - p) % N_DEV,),
                    device_id_type=pl.DeviceIdType.MESH,
                )

        x = x_ref[:, :]
        v = x
        k = 1
        while k < 64:
            shifted = jnp.concatenate(
                [jnp.ones((k, n), v.dtype), v[: m_per - k, :]], axis=0
            )
            v = v * shifted
            k *= 2

        parts = [v[64 * j + 63 : 64 * j + 64, :] for j in range(16)]
        while len(parts) > 1:
            parts = [parts[i] * parts[i + 1] for i in range(0, len(parts), 2)]
        total_ref[:, :] = parts[0]

        for c in range(N_DEV - 1):
            @pl.when(my_pos == c)
            def _(c=c):
                pl.semaphore_wait(barrier_sem, N_DEV - 1 - c)

        for d in range(1, N_DEV):
            @pl.when(my_pos + d < N_DEV)
            def _(d=d):
                pltpu.make_async_remote_copy(
                    src_ref=total_ref,
                    dst_ref=recv_buf.at[d],
                    send_sem=send_sems.at[d],
                    recv_sem=recv_sems.at[d],
                    device_id=((my_pos + d) % N_DEV,),
                    device_id_type=pl.DeviceIdType.MESH,
                ).start()

        while k < m_per:
            shifted = jnp.concatenate(
                [jnp.ones((k, n), v.dtype), v[: m_per - k, :]], axis=0
            )
            v = v * shifted
            k *= 2

        for d in range(1, N_DEV):
            @pl.when(my_pos >= d)
            def _(d=d):
                pltpu.make_async_remote_copy(
                    src_ref=total_ref,
                    dst_ref=recv_buf.at[d],
                    send_sem=send_sems.at[d],
                    recv_sem=recv_sems.at[d],
                    device_id=((my_pos - d) % N_DEV,),
                    device_id_type=pl.DeviceIdType.MESH,
                ).wait_recv()

        prefix = jnp.ones((1, n), jnp.float32)
        for d in range(1, N_DEV):
            prefix = prefix * jnp.where(my_pos >= d, recv_buf[d], 1.0)
        out_ref[:, :] = v * prefix

        for d in range(1, N_DEV):
            @pl.when(my_pos + d < N_DEV)
            def _(d=d):
                pltpu.make_async_remote_copy(
                    src_ref=total_ref,
                    dst_ref=recv_buf.at[d],
                    send_sem=send_sems.at[d],
                    recv_sem=recv_sems.at[d],
                    device_id=((my_pos + d) % N_DEV,),
                    device_id_type=pl.DeviceIdType.MESH,
                ).wait_send()

    return pl.pallas_call(
        body,
        out_shape=jax.ShapeDtypeStruct((m_per, n), x.dtype),
        in_specs=[pl.BlockSpec(memory_space=pltpu.VMEM)],
        out_specs=pl.BlockSpec(memory_space=pltpu.VMEM),
        scratch_shapes=[
            pltpu.VMEM((1, n), x.dtype),
            pltpu.VMEM((N_DEV, 1, n), x.dtype),
            pltpu.SemaphoreType.DMA((N_DEV,)),
            pltpu.SemaphoreType.DMA((N_DEV,)),
        ],
        compiler_params=pltpu.CompilerParams(collective_id=0),
    )(x)
